# baseline (speedup 1.0000x reference)
import jax
import jax.numpy as jnp
from jax import lax
from jax.experimental import pallas as pl
from jax.experimental.pallas import tpu as pltpu
from jax.experimental.pallas import tpu_sc as plsc

_SEQ = 8192
_DIM = 1024

_NC = 2
_NS = 16
_NW = _NC * _NS
_SC_ROWS = 2048
_ROWS_PER_W = _SC_ROWS // _NW
_NBUF = 2
_CH = 32
_NCH = _ROWS_PER_W // _CH

_TC_ROWS = _SEQ - _SC_ROWS
_TC_BLK = 512


def _sc_body(table_hbm, out_hbm, *scratch):
    bufs = scratch[:_NBUF]
    isems = scratch[_NBUF:2 * _NBUF]
    osems = scratch[2 * _NBUF:]
    wid = lax.axis_index("s") * _NC + lax.axis_index("c")
    base = wid * _ROWS_PER_W

    def in_copy(c):
        b = c % _NBUF
        return pltpu.make_async_copy(
            table_hbm.at[pl.ds(base + c * _CH, _CH)], bufs[b], isems[b])

    def out_copy(c):
        b = c % _NBUF
        return pltpu.make_async_copy(
            bufs[b], out_hbm.at[pl.ds(base + c * _CH, _CH)], osems[b])

    for c in range(min(_NBUF, _NCH)):
        in_copy(c).start()
    for c in range(_NCH):
        in_copy(c).wait()
        out_copy(c).start()
        if c + _NBUF < _NCH:
            out_copy(c).wait()
            in_copy(c + _NBUF).start()
    for c in range(max(0, _NCH - _NBUF), _NCH):
        out_copy(c).wait()


def _tc_body(aliased_ref, in_ref, out_ref):
    del aliased_ref
    out_ref[...] = in_ref[...]


def kernel(x, emb_weight):
    del x
    mesh = plsc.VectorSubcoreMesh(core_axis_name="c", subcore_axis_name="s")
    sc_run = pl.kernel(
        _sc_body,
        mesh=mesh,
        out_type=jax.ShapeDtypeStruct((_SEQ, _DIM), jnp.float32),
        scratch_types=(
            [pltpu.VMEM((_CH, _DIM), jnp.float32) for _ in range(_NBUF)]
            + [pltpu.SemaphoreType.DMA for _ in range(2 * _NBUF)]
        ),
    )
    sc_out = sc_run(emb_weight)
    blk0 = _SC_ROWS // _TC_BLK
    return pl.pallas_call(
        _tc_body,
        grid=(_TC_ROWS // _TC_BLK,),
        in_specs=[
            pl.BlockSpec(memory_space=pl.ANY),
            pl.BlockSpec((_TC_BLK, _DIM), lambda i, b0=blk0: (i + b0, 0)),
        ],
        out_specs=pl.BlockSpec((_TC_BLK, _DIM), lambda i, b0=blk0: (i + b0, 0)),
        out_shape=jax.ShapeDtypeStruct((_SEQ, _DIM), jnp.float32),
        input_output_aliases={0: 0},
    )(sc_out, emb_weight)


# speedup vs baseline: 1.6348x; 1.6348x over previous
import jax
import jax.numpy as jnp
from jax.experimental import pallas as pl

_SEQ = 8192
_DIM = 1024
_SC_ROWS = 2048
_TC_ROWS = _SEQ - _SC_ROWS
_TC_BLK = 512


def _head_body(in_ref, out_ref):
    out_ref[...] = in_ref[...]


def _tail_body(aliased_ref, in_ref, out_ref):
    del aliased_ref
    out_ref[...] = in_ref[...]


def kernel(x, emb_weight):
    del x
    head = pl.pallas_call(
        _head_body,
        grid=(_SC_ROWS // _TC_BLK,),
        in_specs=[pl.BlockSpec((_TC_BLK, _DIM), lambda i: (i, 0))],
        out_specs=pl.BlockSpec((_TC_BLK, _DIM), lambda i: (i, 0)),
        out_shape=jax.ShapeDtypeStruct((_SEQ, _DIM), jnp.float32),
    )(emb_weight)
    blk0 = _SC_ROWS // _TC_BLK
    return pl.pallas_call(
        _tail_body,
        grid=(_TC_ROWS // _TC_BLK,),
        in_specs=[
            pl.BlockSpec(memory_space=pl.ANY),
            pl.BlockSpec((_TC_BLK, _DIM), lambda i, b0=blk0: (i + b0, 0)),
        ],
        out_specs=pl.BlockSpec((_TC_BLK, _DIM), lambda i, b0=blk0: (i + b0, 0)),
        out_shape=jax.ShapeDtypeStruct((_SEQ, _DIM), jnp.float32),
        input_output_aliases={0: 0},
    )(head, emb_weight)
